# Initial kernel scaffold; baseline (speedup 1.0000x reference)
import jax
import jax.numpy as jnp
from jax.experimental import pallas as pl


def kernel(x, edge_index, edge_attr, batch, Wq, bq, Wk, bk, Wv, bv, We, be, Wskip, bskip, Wlin, blin):
    raise NotImplementedError("write your pallas kernel here")


# speedup vs baseline: 5.7772x; 5.7772x over previous
import jax
import jax.numpy as jnp
from jax import lax
from jax.experimental import pallas as pl
from jax.experimental.pallas import tpu as pltpu
from jax.experimental.pallas import tpu_sc as plsc

N = 10000
E = 320000
D = 128
H = 128
O = 64
ED = 16
G = 128

NC = 2
NS = 16
NW = NC * NS
EW = E // NW
C = 32
NCHUNKG = E // C
NITER = (NCHUNKG + NW - 1) // NW
NITER8 = ((NITER + 7) // 8) * 8
NP = 10240
RT = NP // NS
RB = C
NP4 = NP // 4
RT4 = NP4 // NS
SCALE = 1.0 / (H ** 0.5)


def _proj_body(x_ref, w4_ref, b4_ref, wet_ref, dstt_ref, kv_ref, sk_ref):
    xb = x_ref[...]
    h = jnp.dot(xb, w4_ref[...], preferred_element_type=jnp.float32) + b4_ref[...]
    q = h[:, 0:128]
    dstt_ref[:, 0:128] = q
    dstt_ref[:, 128:144] = jnp.dot(q, wet_ref[...], preferred_element_type=jnp.float32)
    dstt_ref[:, 144:256] = jnp.zeros((xb.shape[0], 112), jnp.float32)
    kv_ref[:, 0:128] = h[:, 128:256]
    kv_ref[:, 128:256] = h[:, 256:384]
    sk_ref[...] = h[:, 384:512]


def _project(x, w4, b4, wet):
    bn = 1000
    grid = (N // bn,)
    return pl.pallas_call(
        _proj_body,
        grid=grid,
        in_specs=[
            pl.BlockSpec((bn, D), lambda i: (i, 0)),
            pl.BlockSpec((D, 4 * H), lambda i: (0, 0)),
            pl.BlockSpec((1, 4 * H), lambda i: (0, 0)),
            pl.BlockSpec((D, ED), lambda i: (0, 0)),
        ],
        out_specs=[
            pl.BlockSpec((bn, 2 * H), lambda i: (i, 0)),
            pl.BlockSpec((bn, 2 * H), lambda i: (i, 0)),
            pl.BlockSpec((bn, H), lambda i: (i, 0)),
        ],
        out_shape=[
            jax.ShapeDtypeStruct((N, 2 * H), jnp.float32),
            jax.ShapeDtypeStruct((N, 2 * H), jnp.float32),
            jax.ShapeDtypeStruct((N, H), jnp.float32),
        ],
    )(x, w4, b4, wet)


def _edge_body(dstt_hbm, kv_hbm, reci_hbm, recf_hbm, out_hbm, out2_hbm,
               acc, acc2,
               riBig, reBig, sidxA, sidxB, didxA, didxB, d2A, d2B,
               qb, kvb, ob, ob2, semG, semS):
    c = lax.axis_index("c")
    s = lax.axis_index("s")
    zv = jnp.zeros((16,), jnp.float32)
    lane = lax.iota(jnp.int32, 16)
    dn = lax.GatherDimensionNumbers(
        offset_dims=(), collapsed_slice_dims=(0,), start_index_map=(0,))

    def _zrow(r, carry):
        for j in range(8):
            ob[r, pl.ds(16 * j, 16)] = zv
            ob2[r, pl.ds(16 * j, 16)] = zv
        return carry
    lax.fori_loop(0, RB, _zrow, 0)
    rbase = s * RT
    for i in range(RT // RB):
        pltpu.sync_copy(ob, acc.at[pl.ds(rbase + i * RB, RB)])
    r2base = s * RT4
    for i in range(RT4 // RB):
        pltpu.sync_copy(ob, acc2.at[pl.ds(r2base + i * RB, RB)])
    plsc.subcore_barrier()

    wid = c * NS + s

    def _extract(ro, SIDX, DIDX, D2):
        for i in range(C // 16):
            SIDX[pl.ds(16 * i, 16)] = riBig[ro, pl.ds(16 * i, 16)]
            dv = riBig[ro, pl.ds(32 + 16 * i, 16)]
            DIDX[pl.ds(16 * i, 16)] = dv
            D2[pl.ds(16 * i, 16)] = dv >> 2

    def _gissue(SIDX, DIDX):
        pltpu.async_copy(dstt_hbm.at[DIDX.at[pl.ds(0, C)]], qb, semG)
        pltpu.async_copy(kv_hbm.at[SIDX], kvb, semG)

    def _gdrain(SIDX, DIDX):
        pltpu.make_async_copy(dstt_hbm.at[DIDX.at[pl.ds(0, C)]], qb, semG).wait()
        pltpu.make_async_copy(kv_hbm.at[SIDX], kvb, semG).wait()

    CH = C // 2

    def _sissue(DIDX, D2, o):
        pltpu.async_copy(ob.at[pl.ds(o, CH)],
                         acc.at[DIDX.at[pl.ds(o, CH)]], semS, add=True)
        pltpu.async_copy(ob2.at[pl.ds(o, CH)],
                         acc2.at[D2.at[pl.ds(o, CH)]], semS, add=True)

    def _sissue2(DIDX, D2):
        pltpu.async_copy(ob, acc.at[DIDX.at[pl.ds(0, C)]], semS, add=True)
        pltpu.async_copy(ob2, acc2.at[D2.at[pl.ds(0, C)]], semS, add=True)

    def _sdrain(DIDX, D2):
        pltpu.make_async_copy(ob, acc.at[DIDX.at[pl.ds(0, C)]], semS).wait()
        pltpu.make_async_copy(ob2, acc2.at[D2.at[pl.ds(0, C)]], semS).wait()

    def _compute(ro, DIDX, D2, rlo, rhi):
        def _grp(r, carry):
            for j in range(8):
                e = r * 8 + j
                eav = reBig[4 * ro + r, pl.ds(16 * j, 16)]
                accv = qb[e, pl.ds(128, 16)] * eav
                for hh in range(8):
                    accv = accv + qb[e, pl.ds(16 * hh, 16)] * kvb[e, pl.ds(16 * hh, 16)]
                for sh in (8, 4, 2, 1):
                    accv = accv + lax.gather(
                        accv, (lane ^ sh)[:, None], dn, slice_sizes=(1,),
                        mode=lax.GatherScatterMode.PROMISE_IN_BOUNDS)
                wv = jnp.exp(jnp.minimum(accv * SCALE, 60.0))
                for hh in range(8):
                    ob[e, pl.ds(16 * hh, 16)] = kvb[e, pl.ds(128 + 16 * hh, 16)] * wv
                d = DIDX[pl.ds(e, 16)][0]
                goff = pl.multiple_of((d & 3) * 32, 32)
                ob2[e, pl.ds(goff, 16)] = eav * wv
                ob2[e, pl.ds(goff + 16, 16)] = jnp.where(lane == 0, wv, zv)
            return carry
        lax.fori_loop(rlo, rhi, _grp, 0)

    def _cleanup(DIDX):
        def _cl(e, carry):
            d = DIDX[pl.ds(e, 16)][0]
            goff = pl.multiple_of((d & 3) * 32, 32)
            ob2[e, pl.ds(goff, 16)] = zv
            ob2[e, pl.ds(goff + 16, 16)] = zv
            return carry
        lax.fori_loop(0, C, _cl, 0)

    A = (sidxA, didxA, d2A)
    B = (sidxB, didxB, d2B)

    def _half(jv, CUR, NXT):
        cSIDX, cDIDX, cD2 = CUR
        nSIDX, nDIDX, nD2 = NXT
        cid = jv * NW + wid
        ro = jv & 7

        @pl.when(jnp.logical_and(ro == 0, cid < NCHUNKG))
        def _wg():
            g8 = (jv >> 3) * 8
            pltpu.sync_copy(reci_hbm.at[wid, pl.ds(g8, 8)], riBig)
            _extract(ro, cSIDX, cDIDX, cD2)
            _gissue(cSIDX, cDIDX)

        @pl.when(jnp.logical_and((jv & 3) == 0, cid < NCHUNKG))
        def _wge():
            pltpu.sync_copy(recf_hbm.at[wid, pl.ds((jv >> 2) * 16, 16)], reBig)

        @pl.when(jnp.logical_and(jv >= 1, cid - NW < NCHUNKG))
        def _w2():
            _sdrain(nDIDX, nD2)
            _cleanup(nDIDX)

        @pl.when(cid < NCHUNKG)
        def _w4():
            _gdrain(cSIDX, cDIDX)
            _compute(jv & 3, cDIDX, cD2, 0, C // 8)
            _sissue2(cDIDX, cD2)

        @pl.when(jnp.logical_and((jv + 1) & 7 != 0, cid + NW < NCHUNKG))
        def _w5():
            _extract((jv + 1) & 7, nSIDX, nDIDX, nD2)
            _gissue(nSIDX, nDIDX)

    def _pair(t, carry):
        _half(2 * t, A, B)
        _half(2 * t + 1, B, A)
        return carry
    lax.fori_loop(0, (NITER + 1) // 2, _pair, 0)

    plsc.subcore_barrier()
    for i in range(RT // RB):
        r0 = rbase + i * RB
        pltpu.sync_copy(acc.at[pl.ds(r0, RB)], ob)
        pltpu.sync_copy(ob, out_hbm.at[c, pl.ds(r0, RB)])
    for i in range(RT4 // RB):
        r0 = r2base + i * RB
        pltpu.sync_copy(acc2.at[pl.ds(r0, RB)], ob)
        pltpu.sync_copy(ob, out2_hbm.at[c, pl.ds(r0, RB)])


def _edge_pass(dstt, kv, reci, recf):
    mesh = plsc.VectorSubcoreMesh(core_axis_name="c", subcore_axis_name="s")
    fn = pl.kernel(
        _edge_body,
        out_type=[
            jax.ShapeDtypeStruct((NC, NP, H), jnp.float32),
            jax.ShapeDtypeStruct((NC, NP4, 128), jnp.float32),
        ],
        mesh=mesh,
        scratch_types=[
            pltpu.VMEM_SHARED((NP, H), jnp.float32),
            pltpu.VMEM_SHARED((NP4, 128), jnp.float32),
            pltpu.VMEM((8, 128), jnp.int32),
            pltpu.VMEM((16, 128), jnp.float32),
            pltpu.VMEM((C,), jnp.int32),
            pltpu.VMEM((C,), jnp.int32),
            pltpu.VMEM((C + 16,), jnp.int32),
            pltpu.VMEM((C + 16,), jnp.int32),
            pltpu.VMEM((C + 16,), jnp.int32),
            pltpu.VMEM((C + 16,), jnp.int32),
            pltpu.VMEM((C, 2 * H), jnp.float32),
            pltpu.VMEM((C, 2 * H), jnp.float32),
            pltpu.VMEM((C, H), jnp.float32),
            pltpu.VMEM((C, 128), jnp.float32),
            pltpu.SemaphoreType.DMA,
            pltpu.SemaphoreType.DMA,
        ],
    )
    return fn(dstt, kv, reci, recf)


def _finish_body(acc_ref, acc2_ref, sk_ref, bat_ref, we_ref, be_ref,
                 wl_ref, bl_ref, out_ref, sums, counts):
    i = pl.program_id(0)

    @pl.when(i == 0)
    def _init():
        sums[...] = jnp.zeros_like(sums)
        counts[...] = jnp.zeros_like(counts)

    num = acc_ref[0] + acc_ref[1]
    a2 = acc2_ref[0] + acc2_ref[1]
    aea = a2[:, 0:16]
    den = a2[:, 16:17]
    inv = 1.0 / (den + 1e-16)
    conv = (num + jnp.dot(aea, we_ref[...], preferred_element_type=jnp.float32)) * inv
    conv = conv + (den * inv) * be_ref[...]
    hrow = jnp.maximum(conv + sk_ref[...], 0.0)

    bids = bat_ref[0, 0, :]
    gids = lax.broadcasted_iota(jnp.int32, (G, bids.shape[0]), 0)
    mask = (gids == bids[None, :]).astype(jnp.float32)
    sums[...] += jnp.dot(mask, hrow, preferred_element_type=jnp.float32)
    counts[...] += jnp.sum(mask, axis=1, keepdims=True)

    @pl.when(i == pl.num_programs(0) - 1)
    def _fin():
        pooled = sums[...] / jnp.maximum(counts[...], 1.0)
        out_ref[...] = jnp.dot(pooled, wl_ref[...], preferred_element_type=jnp.float32) + bl_ref[...]


def _finish(acc, acc2r, sk, batch3, we, be, wl, bl):
    bn = 1000
    grid = (N // bn,)
    return pl.pallas_call(
        _finish_body,
        grid=grid,
        in_specs=[
            pl.BlockSpec((NC, bn, H), lambda i: (0, i, 0)),
            pl.BlockSpec((NC, bn, 32), lambda i: (0, i, 0)),
            pl.BlockSpec((bn, H), lambda i: (i, 0)),
            pl.BlockSpec((1, 1, bn), lambda i: (i, 0, 0)),
            pl.BlockSpec((ED, H), lambda i: (0, 0)),
            pl.BlockSpec((1, H), lambda i: (0, 0)),
            pl.BlockSpec((H, O), lambda i: (0, 0)),
            pl.BlockSpec((1, O), lambda i: (0, 0)),
        ],
        out_specs=pl.BlockSpec((G, O), lambda i: (0, 0)),
        out_shape=jax.ShapeDtypeStruct((G, O), jnp.float32),
        scratch_shapes=[
            pltpu.VMEM((G, H), jnp.float32),
            pltpu.VMEM((G, 1), jnp.float32),
        ],
    )(acc, acc2r, sk, batch3, we, be, wl, bl)


@jax.jit
def kernel(x, edge_index, edge_attr, batch, Wq, bq, Wk, bk, Wv, bv, We, be,
           Wskip, bskip, Wlin, blin):
    src = edge_index[0].astype(jnp.int32)
    dst = edge_index[1].astype(jnp.int32)
    w4 = jnp.concatenate([Wq, Wk, Wv, Wskip], axis=1)
    b4 = jnp.concatenate([bq, bk, bv, bskip]).reshape(1, 4 * H)
    wet = We.T

    dstt, kv, sk = _project(x, w4, b4, wet)
    reci0 = jnp.concatenate([
        src.reshape(NCHUNKG, 1, C),
        dst.reshape(NCHUNKG, 1, C),
        jnp.zeros((NCHUNKG, 1, 128 - 2 * C), jnp.int32)], axis=2)
    pad_c = NITER8 * NW - NCHUNKG
    reci = jnp.concatenate(
        [reci0, jnp.zeros((pad_c, 1, 128), jnp.int32)], axis=0)
    reci = reci.reshape(NITER8, NW, 128).transpose(1, 0, 2)
    recf0 = edge_attr.reshape(NCHUNKG, 4, 128)
    recf = jnp.concatenate(
        [recf0, jnp.zeros((pad_c, 4, 128), jnp.float32)], axis=0)
    recf = recf.reshape(NITER8, NW, 4, 128).transpose(1, 0, 2, 3)
    recf = recf.reshape(NW, NITER8 * 4, 128)
    acc, acc2 = _edge_pass(dstt, kv, reci, recf)
    acc2r = acc2.reshape(NC, NP, 32)
    batch3 = batch.astype(jnp.int32).reshape(N // 1000, 1, 1000)
    return _finish(acc, acc2r, sk, batch3, We, be.reshape(1, H),
                   Wlin, blin.reshape(1, O))


# speedup vs baseline: 5.9555x; 1.0309x over previous
import jax
import jax.numpy as jnp
from jax import lax
from jax.experimental import pallas as pl
from jax.experimental.pallas import tpu as pltpu
from jax.experimental.pallas import tpu_sc as plsc

N = 10000
E = 320000
D = 128
H = 128
O = 64
ED = 16
G = 128

NC = 2
NS = 16
NW = NC * NS
EW = E // NW
C = 32
NCHUNKG = E // C
NITER = (NCHUNKG + NW - 1) // NW
NP = 10240
RT = NP // NS
RB = C
NP4 = NP // 4
RT4 = NP4 // NS
SCALE = 1.0 / (H ** 0.5)


def _proj_body(x_ref, w4_ref, b4_ref, wet_ref, tq_ref, sk_ref):
    xb = x_ref[...]
    h = jnp.dot(xb, w4_ref[...], preferred_element_type=jnp.float32) + b4_ref[...]
    q = h[:, 0:128]
    tq_ref[0, :, 0:128] = q
    tq_ref[0, :, 128:144] = jnp.dot(q, wet_ref[...], preferred_element_type=jnp.float32)
    tq_ref[0, :, 144:256] = jnp.zeros((xb.shape[0], 112), jnp.float32)
    tq_ref[1, :, 0:128] = h[:, 128:256]
    tq_ref[1, :, 128:256] = h[:, 256:384]
    sk_ref[...] = h[:, 384:512]


def _project(x, w4, b4, wet):
    bn = 1000
    grid = (N // bn,)
    return pl.pallas_call(
        _proj_body,
        grid=grid,
        in_specs=[
            pl.BlockSpec((bn, D), lambda i: (i, 0)),
            pl.BlockSpec((D, 4 * H), lambda i: (0, 0)),
            pl.BlockSpec((1, 4 * H), lambda i: (0, 0)),
            pl.BlockSpec((D, ED), lambda i: (0, 0)),
        ],
        out_specs=[
            pl.BlockSpec((2, bn, 2 * H), lambda i: (0, i, 0)),
            pl.BlockSpec((bn, H), lambda i: (i, 0)),
        ],
        out_shape=[
            jax.ShapeDtypeStruct((2, N, 2 * H), jnp.float32),
            jax.ShapeDtypeStruct((N, H), jnp.float32),
        ],
    )(x, w4, b4, wet)


def _edge_body(tq_hbm, rec_hbm, out_hbm, out2_hbm,
               acc, acc2,
               recA, recB, cidxA, cidxB, didxA, didxB, d2A, d2B,
               qkv, ob, ob2, semG, semS, semI):
    c = lax.axis_index("c")
    s = lax.axis_index("s")
    zv = jnp.zeros((16,), jnp.float32)
    lane = lax.iota(jnp.int32, 16)
    dn = lax.GatherDimensionNumbers(
        offset_dims=(), collapsed_slice_dims=(0,), start_index_map=(0,))

    def _zrow(r, carry):
        for j in range(8):
            ob[r, pl.ds(16 * j, 16)] = zv
            ob2[r, pl.ds(16 * j, 16)] = zv
        return carry
    lax.fori_loop(0, RB, _zrow, 0)
    rbase = s * RT
    for i in range(RT // RB):
        pltpu.sync_copy(ob, acc.at[pl.ds(rbase + i * RB, RB)])
    r2base = s * RT4
    for i in range(RT4 // RB):
        pltpu.sync_copy(ob, acc2.at[pl.ds(r2base + i * RB, RB)])
    plsc.subcore_barrier()

    wid = c * NS + s

    def _extract(REC, CIDX, DIDX, D2):
        for i in range(C // 16):
            sv = REC[0, pl.ds(16 * i, 16)].astype(jnp.int32)
            dv = REC[0, pl.ds(32 + 16 * i, 16)].astype(jnp.int32)
            CIDX[pl.ds(16 * i, 16)] = dv
            CIDX[pl.ds(C + 16 * i, 16)] = sv + N
            DIDX[pl.ds(16 * i, 16)] = dv
            D2[pl.ds(16 * i, 16)] = dv >> 2

    def _gissue(CIDX):
        pltpu.async_copy(tq_hbm.at[CIDX], qkv, semG)

    def _gdrain(CIDX):
        pltpu.make_async_copy(tq_hbm.at[CIDX], qkv, semG).wait()

    CH = C // 2

    def _sissue(DIDX, D2, o):
        pltpu.async_copy(ob.at[pl.ds(o, CH)],
                         acc.at[DIDX.at[pl.ds(o, CH)]], semS, add=True)
        pltpu.async_copy(ob2.at[pl.ds(o, CH)],
                         acc2.at[D2.at[pl.ds(o, CH)]], semS, add=True)

    def _sissue2(DIDX, D2):
        pltpu.async_copy(ob, acc.at[DIDX.at[pl.ds(0, C)]], semS, add=True)
        pltpu.async_copy(ob2, acc2.at[D2.at[pl.ds(0, C)]], semS, add=True)

    def _sdrain(DIDX, D2):
        pltpu.make_async_copy(ob, acc.at[DIDX.at[pl.ds(0, C)]], semS).wait()
        pltpu.make_async_copy(ob2, acc2.at[D2.at[pl.ds(0, C)]], semS).wait()

    def _compute(RE, DIDX, D2, rlo, rhi):
        def _grp(r, carry):
            for j in range(8):
                e = r * 8 + j
                eav = RE[1 + r, pl.ds(16 * j, 16)]
                accv = qkv[e, pl.ds(128, 16)] * eav
                for hh in range(8):
                    accv = accv + qkv[e, pl.ds(16 * hh, 16)] * qkv[C + e, pl.ds(16 * hh, 16)]
                for sh in (8, 4, 2, 1):
                    accv = accv + lax.gather(
                        accv, (lane ^ sh)[:, None], dn, slice_sizes=(1,),
                        mode=lax.GatherScatterMode.PROMISE_IN_BOUNDS)
                wv = jnp.exp(jnp.minimum(accv * SCALE, 60.0))
                for hh in range(8):
                    ob[e, pl.ds(16 * hh, 16)] = qkv[C + e, pl.ds(128 + 16 * hh, 16)] * wv
                d = DIDX[pl.ds(e, 16)][0]
                goff = pl.multiple_of((d & 3) * 32, 32)
                ob2[e, pl.ds(goff, 16)] = eav * wv
                ob2[e, pl.ds(goff + 16, 16)] = jnp.where(lane == 0, wv, zv)
            return carry
        lax.fori_loop(rlo, rhi, _grp, 0)

    def _cleanup(DIDX):
        def _cl(e, carry):
            d = DIDX[pl.ds(e, 16)][0]
            goff = pl.multiple_of((d & 3) * 32, 32)
            ob2[e, pl.ds(goff, 16)] = zv
            ob2[e, pl.ds(goff + 16, 16)] = zv
            return carry
        lax.fori_loop(0, C, _cl, 0)

    A = (recA, cidxA, didxA, d2A)
    B = (recB, cidxB, didxB, d2B)

    def _half(jv, CUR, NXT):
        cREC, cCIDX, cDIDX, cD2 = CUR
        nREC, nCIDX, nDIDX, nD2 = NXT
        cid = jv * NW + wid

        @pl.when(cid + NW < NCHUNKG)
        def _w3():
            pltpu.async_copy(rec_hbm.at[cid + NW], nREC, semI)

        @pl.when(jnp.logical_and(jv >= 1, cid - NW < NCHUNKG))
        def _w2():
            _sdrain(nDIDX, nD2)
            _cleanup(nDIDX)

        @pl.when(cid < NCHUNKG)
        def _w4():
            _gdrain(cCIDX)
            _compute(cREC, cDIDX, cD2, 0, C // 8)
            _sissue2(cDIDX, cD2)

        @pl.when(cid + NW < NCHUNKG)
        def _w5():
            pltpu.make_async_copy(rec_hbm.at[cid + NW], nREC, semI).wait()
            _extract(nREC, nCIDX, nDIDX, nD2)
            _gissue(nCIDX)

    pltpu.async_copy(rec_hbm.at[wid], recA, semI).wait()
    _extract(recA, cidxA, didxA, d2A)
    _gissue(cidxA)

    def _pair(t, carry):
        _half(2 * t, A, B)
        _half(2 * t + 1, B, A)
        return carry
    lax.fori_loop(0, (NITER + 1) // 2, _pair, 0)

    plsc.subcore_barrier()
    for i in range(RT // RB):
        r0 = rbase + i * RB
        pltpu.sync_copy(acc.at[pl.ds(r0, RB)], ob)
        pltpu.sync_copy(ob, out_hbm.at[c, pl.ds(r0, RB)])
    for i in range(RT4 // RB):
        r0 = r2base + i * RB
        pltpu.sync_copy(acc2.at[pl.ds(r0, RB)], ob)
        pltpu.sync_copy(ob, out2_hbm.at[c, pl.ds(r0, RB)])


def _edge_pass(tq, rec):
    mesh = plsc.VectorSubcoreMesh(core_axis_name="c", subcore_axis_name="s")
    fn = pl.kernel(
        _edge_body,
        out_type=[
            jax.ShapeDtypeStruct((NC, NP, H), jnp.float32),
            jax.ShapeDtypeStruct((NC, NP4, 128), jnp.float32),
        ],
        mesh=mesh,
        scratch_types=[
            pltpu.VMEM_SHARED((NP, H), jnp.float32),
            pltpu.VMEM_SHARED((NP4, 128), jnp.float32),
            pltpu.VMEM((5, 128), jnp.float32),
            pltpu.VMEM((5, 128), jnp.float32),
            pltpu.VMEM((2 * C,), jnp.int32),
            pltpu.VMEM((2 * C,), jnp.int32),
            pltpu.VMEM((C + 16,), jnp.int32),
            pltpu.VMEM((C + 16,), jnp.int32),
            pltpu.VMEM((C + 16,), jnp.int32),
            pltpu.VMEM((C + 16,), jnp.int32),
            pltpu.VMEM((2 * C, 2 * H), jnp.float32),
            pltpu.VMEM((C, H), jnp.float32),
            pltpu.VMEM((C, 128), jnp.float32),
            pltpu.SemaphoreType.DMA,
            pltpu.SemaphoreType.DMA,
            pltpu.SemaphoreType.DMA,
        ],
    )
    return fn(tq, rec)


def _finish_body(acc_ref, acc2_ref, sk_ref, bat_ref, we_ref, be_ref,
                 wl_ref, bl_ref, out_ref, sums, counts):
    i = pl.program_id(0)

    @pl.when(i == 0)
    def _init():
        sums[...] = jnp.zeros_like(sums)
        counts[...] = jnp.zeros_like(counts)

    num = acc_ref[0] + acc_ref[1]
    a2 = acc2_ref[0] + acc2_ref[1]
    aea = a2[:, 0:16]
    den = a2[:, 16:17]
    inv = 1.0 / (den + 1e-16)
    conv = (num + jnp.dot(aea, we_ref[...], preferred_element_type=jnp.float32)) * inv
    conv = conv + (den * inv) * be_ref[...]
    hrow = jnp.maximum(conv + sk_ref[...], 0.0)

    bids = bat_ref[0, 0, :]
    gids = lax.broadcasted_iota(jnp.int32, (G, bids.shape[0]), 0)
    mask = (gids == bids[None, :]).astype(jnp.float32)
    sums[...] += jnp.dot(mask, hrow, preferred_element_type=jnp.float32)
    counts[...] += jnp.sum(mask, axis=1, keepdims=True)

    @pl.when(i == pl.num_programs(0) - 1)
    def _fin():
        pooled = sums[...] / jnp.maximum(counts[...], 1.0)
        out_ref[...] = jnp.dot(pooled, wl_ref[...], preferred_element_type=jnp.float32) + bl_ref[...]


def _finish(acc, acc2r, sk, batch3, we, be, wl, bl):
    bn = 1000
    grid = (N // bn,)
    return pl.pallas_call(
        _finish_body,
        grid=grid,
        in_specs=[
            pl.BlockSpec((NC, bn, H), lambda i: (0, i, 0)),
            pl.BlockSpec((NC, bn, 32), lambda i: (0, i, 0)),
            pl.BlockSpec((bn, H), lambda i: (i, 0)),
            pl.BlockSpec((1, 1, bn), lambda i: (i, 0, 0)),
            pl.BlockSpec((ED, H), lambda i: (0, 0)),
            pl.BlockSpec((1, H), lambda i: (0, 0)),
            pl.BlockSpec((H, O), lambda i: (0, 0)),
            pl.BlockSpec((1, O), lambda i: (0, 0)),
        ],
        out_specs=pl.BlockSpec((G, O), lambda i: (0, 0)),
        out_shape=jax.ShapeDtypeStruct((G, O), jnp.float32),
        scratch_shapes=[
            pltpu.VMEM((G, H), jnp.float32),
            pltpu.VMEM((G, 1), jnp.float32),
        ],
    )(acc, acc2r, sk, batch3, we, be, wl, bl)


@jax.jit
def kernel(x, edge_index, edge_attr, batch, Wq, bq, Wk, bk, Wv, bv, We, be,
           Wskip, bskip, Wlin, blin):
    src = edge_index[0].astype(jnp.int32)
    dst = edge_index[1].astype(jnp.int32)
    w4 = jnp.concatenate([Wq, Wk, Wv, Wskip], axis=1)
    b4 = jnp.concatenate([bq, bk, bv, bskip]).reshape(1, 4 * H)
    wet = We.T

    tqkv, sk = _project(x, w4, b4, wet)
    tq = tqkv.reshape(2 * N, 2 * H)
    row0 = jnp.concatenate([
        src.astype(jnp.float32).reshape(NCHUNKG, 1, C),
        dst.astype(jnp.float32).reshape(NCHUNKG, 1, C),
        jnp.zeros((NCHUNKG, 1, 128 - 2 * C), jnp.float32)], axis=2)
    rec = jnp.concatenate([row0, edge_attr.reshape(NCHUNKG, 4, 128)], axis=1)
    acc, acc2 = _edge_pass(tq, rec)
    acc2r = acc2.reshape(NC, NP, 32)
    batch3 = batch.astype(jnp.int32).reshape(N // 1000, 1, 1000)
    return _finish(acc, acc2r, sk, batch3, We, be.reshape(1, H),
                   Wlin, blin.reshape(1, O))
